# Initial kernel scaffold; baseline (speedup 1.0000x reference)
import jax
import jax.numpy as jnp
from jax.experimental import pallas as pl


def kernel(hidden_states, W):
    raise NotImplementedError("write your pallas kernel here")


# speedup vs baseline: 1.3344x; 1.3344x over previous
import functools

import jax
import jax.numpy as jnp
from jax.experimental import pallas as pl

NUM_EXPERTS = 64
TOP_K = 8
TB = 512


def _router_block(x_ref, wt_ref, scores_ref, idx_ref):
    logits = jnp.dot(x_ref[...], wt_ref[...], preferred_element_type=jnp.float32)
    iota = jax.lax.broadcasted_iota(jnp.int32, logits.shape, 1)
    vals = logits
    scores = []
    idxs = []
    for _ in range(TOP_K):
        m = jnp.max(vals, axis=-1, keepdims=True)
        i = jnp.argmax(vals, axis=-1, keepdims=True).astype(jnp.int32)
        scores.append(m)
        idxs.append(i)
        vals = jnp.where(iota == i, -jnp.inf, vals)
    scores_ref[...] = jnp.concatenate(scores, axis=-1)
    idx_ref[...] = jnp.concatenate(idxs, axis=-1)


@functools.partial(jax.jit, static_argnames=())
def kernel(hidden_states, W):
    tokens, hidden = hidden_states.shape
    wt = W.T
    grid = (tokens // TB,)
    scores, idx = pl.pallas_call(
        _router_block,
        grid=grid,
        in_specs=[
            pl.BlockSpec((TB, hidden), lambda i: (i, 0)),
            pl.BlockSpec((hidden, NUM_EXPERTS), lambda i: (0, 0)),
        ],
        out_specs=[
            pl.BlockSpec((TB, TOP_K), lambda i: (i, 0)),
            pl.BlockSpec((TB, TOP_K), lambda i: (i, 0)),
        ],
        out_shape=[
            jax.ShapeDtypeStruct((tokens, TOP_K), jnp.float32),
            jax.ShapeDtypeStruct((tokens, TOP_K), jnp.int32),
        ],
    )(hidden_states, wt)
    return scores, idx
